# baseline (speedup 1.0000x reference)
import functools

import jax
import jax.numpy as jnp
import numpy as np
from jax import lax
from jax.experimental import pallas as pl
from jax.experimental.pallas import tpu as pltpu
from jax.experimental.pallas import tpu_sc as plsc

FIELD = 26
LAT = 64
WIDE = 128
FLATW = FIELD * WIDE
EMBED_OUT = FIELD * LAT
PAIR = FIELD * (FIELD - 1) // 2
NOFF = FIELD // 2
BB = 512
CHUNK = 128


def _make_sc_gather(n_rows):
    info = plsc.get_sparse_core_info()
    nw = info.num_cores * info.num_subcores
    chunks_w = n_rows // (nw * CHUNK)
    half = chunks_w // 2
    mesh = plsc.VectorSubcoreMesh(core_axis_name="c", subcore_axis_name="s")

    @functools.partial(
        pl.kernel, mesh=mesh,
        out_type=jax.ShapeDtypeStruct((n_rows, WIDE), jnp.float32),
        scratch_types=[
            pltpu.VMEM((chunks_w, CHUNK), jnp.int32),
            pltpu.VMEM((2, CHUNK, WIDE), jnp.float32),
            pltpu.SemaphoreType.DMA,
            pltpu.SemaphoreType.DMA,
        ],
    )
    def sc_gather(table_hbm, idx_hbm, out_hbm, idx_v, rows_v, g0, g1):
        wid = lax.axis_index("s") * info.num_cores + lax.axis_index("c")
        rbase = wid * chunks_w * CHUNK
        pltpu.sync_copy(idx_hbm.at[wid], idx_v)

        def start(j, slot, sem):
            pltpu.async_copy(table_hbm.at[idx_v.at[j]], rows_v.at[slot], sem)

        def wait(slot, sem):
            pltpu.make_async_copy(
                table_hbm.at[idx_v.at[0]], rows_v.at[slot], sem).wait()

        def store(j, slot):
            pltpu.sync_copy(rows_v.at[slot],
                            out_hbm.at[pl.ds(rbase + j * CHUNK, CHUNK)])

        start(0, 0, g0)

        def body(g, carry):
            j0 = 2 * g
            start(j0 + 1, 1, g1)
            wait(0, g0)
            store(j0, 0)

            @pl.when(j0 + 2 < chunks_w)
            def _():
                start(j0 + 2, 0, g0)

            wait(1, g1)
            store(j0 + 1, 1)
            return carry

        lax.fori_loop(0, half, body, 0)
        if chunks_w % 2:
            wait(0, g0)
            store(chunks_w - 1, 0)

    return sc_gather


def _mlp_body(ab_ref, xv_ref, expc_ref, e_ref, w1a_ref, v_ref, b1_ref,
              w2_ref, b2_ref, w3_ref, b3_ref, wo_ref, bo_ref, out_ref):
    a = ab_ref[...]
    m = jnp.max(a)
    e = jnp.exp(a - m)
    p = e / jnp.sum(e)
    probrep = jnp.dot(p, expc_ref[...], preferred_element_type=jnp.float32)
    xe32 = xv_ref[...] * probrep
    xe = xe32.astype(jnp.bfloat16)
    acc = jnp.dot(xe, w1a_ref[...], preferred_element_type=jnp.float32)
    for o in range(1, NOFF + 1):
        s = WIDE * o
        ro = jnp.concatenate([xe32[:, s:], xe32[:, :s]], axis=1)
        po = (xe32 * ro).astype(jnp.bfloat16)
        q = jnp.dot(po, e_ref[...], preferred_element_type=jnp.float32)
        acc = acc + jnp.dot(q.astype(jnp.bfloat16), v_ref[o - 1],
                            preferred_element_type=jnp.float32)
    h = jnp.maximum(acc + b1_ref[...], 0.0)
    h = jnp.maximum(
        jnp.dot(h.astype(jnp.bfloat16), w2_ref[...],
                preferred_element_type=jnp.float32) + b2_ref[...], 0.0)
    h = jnp.maximum(
        jnp.dot(h.astype(jnp.bfloat16), w3_ref[...],
                preferred_element_type=jnp.float32) + b3_ref[...], 0.0)
    out_ref[...] = jnp.dot(h.astype(jnp.bfloat16), wo_ref[...],
                           preferred_element_type=jnp.float32) + bo_ref[...]


def _mlp_call(ab, xv2d, expc, ematc, w1a, v, b1, W2, b2, W3, b3, Wo, bo,
              *, interpret=False):
    batch = xv2d.shape[0]
    grid = (batch // BB,)
    full = lambda shape: pl.BlockSpec(shape, lambda i: (0,) * len(shape))
    return pl.pallas_call(
        _mlp_body,
        grid=grid,
        in_specs=[
            full((1, 32)),
            pl.BlockSpec((BB, FLATW), lambda i: (i, 0)),
            full(expc.shape), full(ematc.shape), full(w1a.shape),
            full(v.shape), full((1, b1.shape[1])),
            full(W2.shape), full((1, W2.shape[1])),
            full(W3.shape), full((1, W3.shape[1])),
            full(Wo.shape), full((1, 1)),
        ],
        out_specs=pl.BlockSpec((BB, 1), lambda i: (i, 0)),
        out_shape=jax.ShapeDtypeStruct((batch, 1), jnp.float32),
        interpret=interpret,
    )(ab, xv2d, expc, ematc, w1a, v, b1, W2, b2, W3, b3, Wo, bo)


def _pair_index_table():
    pos = np.zeros((FIELD, FIELD), dtype=np.int32)
    rows, cols = np.triu_indices(FIELD, k=1)
    for k, (i, j) in enumerate(zip(rows, cols)):
        pos[i, j] = k
        pos[j, i] = k
    pidx = np.zeros((NOFF, 32), dtype=np.int32)
    mask = np.zeros((NOFF, 32), dtype=np.float32)
    for o in range(1, NOFF + 1):
        for f in range(FIELD):
            g = (f + o) % FIELD
            pidx[o - 1, f] = pos[min(f, g), max(f, g)]
            mask[o - 1, f] = 0.5 if o == NOFF else 1.0
    return pidx, mask


_PIDX, _PMASK = _pair_index_table()

_EMAT = np.zeros((FLATW, 32), dtype=np.float32)
for _f in range(FIELD):
    _EMAT[_f * WIDE:(_f + 1) * WIDE, _f] = 1.0

_EXPC = np.ascontiguousarray(_EMAT.T)


def kernel(x, beta, arch, embedding, W1, b1, W2, b2, W3, b3, Wo, bo):
    batch = x.shape[0]
    nsplit = 2
    bsp = batch // nsplit
    info = plsc.get_sparse_core_info()
    nw = info.num_cores * info.num_subcores
    table = jnp.concatenate(
        [embedding, jnp.zeros_like(embedding)], axis=1)
    sc = _make_sc_gather(bsp * FIELD)
    xvs = []
    for h in range(nsplit):
        xh = x[h * bsp:(h + 1) * bsp]
        idx3d = xh.reshape(nw, bsp * FIELD // (nw * CHUNK), CHUNK).astype(jnp.int32)
        xvs.append(sc(table, idx3d).reshape(bsp, FLATW))

    ab = jnp.full((1, 32), -1e30, jnp.float32)
    ab = ab.at[0, :FIELD].set((arch / beta).astype(jnp.float32))
    w1a = jnp.zeros((FLATW, W1.shape[1]), jnp.float32)
    w1a = w1a.at[(np.arange(EMBED_OUT) // LAT) * WIDE
                 + (np.arange(EMBED_OUT) % LAT)].set(W1[:EMBED_OUT])
    v = jnp.take(W1, EMBED_OUT + _PIDX.reshape(-1), axis=0)
    v = (v * _PMASK.reshape(-1, 1)).reshape(NOFF, 32, W1.shape[1])

    outs = [
        _mlp_call(
            ab, xv2d, jnp.asarray(_EXPC), jnp.asarray(_EMAT, jnp.bfloat16),
            w1a.astype(jnp.bfloat16), v.astype(jnp.bfloat16), b1.reshape(1, -1),
            W2.astype(jnp.bfloat16), b2.reshape(1, -1),
            W3.astype(jnp.bfloat16), b3.reshape(1, -1),
            Wo.astype(jnp.bfloat16), bo.reshape(1, 1))
        for xv2d in xvs]
    return jnp.concatenate(outs, axis=0)[:, 0]


# speedup vs baseline: 1.1439x; 1.1439x over previous
import functools

import jax
import jax.numpy as jnp
import numpy as np
from jax import lax
from jax.experimental import pallas as pl
from jax.experimental.pallas import tpu as pltpu
from jax.experimental.pallas import tpu_sc as plsc

FIELD = 26
LAT = 64
WIDE = 128
FLATW = FIELD * WIDE
EMBED_OUT = FIELD * LAT
PAIR = FIELD * (FIELD - 1) // 2
NOFF = FIELD // 2
BB = 512
CHUNK = 128


def _make_sc_gather(n_rows):
    info = plsc.get_sparse_core_info()
    nw = info.num_cores * info.num_subcores
    chunks_w = n_rows // (nw * CHUNK)
    half = chunks_w // 2
    mesh = plsc.VectorSubcoreMesh(core_axis_name="c", subcore_axis_name="s")

    @functools.partial(
        pl.kernel, mesh=mesh,
        out_type=jax.ShapeDtypeStruct((n_rows, WIDE), jnp.float32),
        scratch_types=[
            pltpu.VMEM((chunks_w, CHUNK), jnp.int32),
            pltpu.VMEM((2, CHUNK, WIDE), jnp.float32),
            pltpu.SemaphoreType.DMA,
            pltpu.SemaphoreType.DMA,
        ],
    )
    def sc_gather(table_hbm, idx_hbm, out_hbm, idx_v, rows_v, g0, g1):
        wid = lax.axis_index("s") * info.num_cores + lax.axis_index("c")
        rbase = wid * chunks_w * CHUNK
        pltpu.sync_copy(idx_hbm.at[wid], idx_v)

        def start(j, slot, sem):
            pltpu.async_copy(table_hbm.at[idx_v.at[j]], rows_v.at[slot], sem)

        def wait(slot, sem):
            pltpu.make_async_copy(
                table_hbm.at[idx_v.at[0]], rows_v.at[slot], sem).wait()

        def store(j, slot):
            pltpu.sync_copy(rows_v.at[slot],
                            out_hbm.at[pl.ds(rbase + j * CHUNK, CHUNK)])

        start(0, 0, g0)

        def body(g, carry):
            j0 = 2 * g
            start(j0 + 1, 1, g1)
            wait(0, g0)
            store(j0, 0)

            @pl.when(j0 + 2 < chunks_w)
            def _():
                start(j0 + 2, 0, g0)

            wait(1, g1)
            store(j0 + 1, 1)
            return carry

        lax.fori_loop(0, half, body, 0)
        if chunks_w % 2:
            wait(0, g0)
            store(chunks_w - 1, 0)

    return sc_gather


def _mlp_body(ab_ref, xv_ref, expc_ref, e_ref, w1a_ref, v_ref, b1_ref,
              w2_ref, b2_ref, w3_ref, b3_ref, wo_ref, bo_ref, out_ref):
    a = ab_ref[...]
    m = jnp.max(a)
    e = jnp.exp(a - m)
    p = e / jnp.sum(e)
    probrep = jnp.dot(p, expc_ref[...], preferred_element_type=jnp.float32)
    xe = xv_ref[...] * probrep
    acc = jnp.dot(xe, w1a_ref[...], preferred_element_type=jnp.float32)
    for o in range(1, NOFF + 1):
        s = WIDE * o
        ro = jnp.concatenate([xe[:, s:], xe[:, :s]], axis=1)
        po = xe * ro
        q = jnp.dot(po, e_ref[...], preferred_element_type=jnp.float32)
        acc = acc + jnp.dot(q, v_ref[o - 1],
                            preferred_element_type=jnp.float32)
    h = jnp.maximum(acc + b1_ref[...], 0.0)
    h = jnp.maximum(
        jnp.dot(h, w2_ref[...], preferred_element_type=jnp.float32)
        + b2_ref[...], 0.0)
    h = jnp.maximum(
        jnp.dot(h, w3_ref[...], preferred_element_type=jnp.float32)
        + b3_ref[...], 0.0)
    out_ref[...] = jnp.dot(h, wo_ref[...],
                           preferred_element_type=jnp.float32) + bo_ref[...]


def _mlp_call(ab, xv2d, expc, ematc, w1a, v, b1, W2, b2, W3, b3, Wo, bo,
              *, interpret=False):
    batch = xv2d.shape[0]
    grid = (batch // BB,)
    full = lambda shape: pl.BlockSpec(shape, lambda i: (0,) * len(shape))
    return pl.pallas_call(
        _mlp_body,
        grid=grid,
        in_specs=[
            full((1, 32)),
            pl.BlockSpec((BB, FLATW), lambda i: (i, 0)),
            full(expc.shape), full(ematc.shape), full(w1a.shape),
            full(v.shape), full((1, b1.shape[1])),
            full(W2.shape), full((1, W2.shape[1])),
            full(W3.shape), full((1, W3.shape[1])),
            full(Wo.shape), full((1, 1)),
        ],
        out_specs=pl.BlockSpec((BB, 1), lambda i: (i, 0)),
        out_shape=jax.ShapeDtypeStruct((batch, 1), jnp.float32),
        interpret=interpret,
    )(ab, xv2d, expc, ematc, w1a, v, b1, W2, b2, W3, b3, Wo, bo)


def _pair_index_table():
    pos = np.zeros((FIELD, FIELD), dtype=np.int32)
    rows, cols = np.triu_indices(FIELD, k=1)
    for k, (i, j) in enumerate(zip(rows, cols)):
        pos[i, j] = k
        pos[j, i] = k
    pidx = np.zeros((NOFF, 32), dtype=np.int32)
    mask = np.zeros((NOFF, 32), dtype=np.float32)
    for o in range(1, NOFF + 1):
        for f in range(FIELD):
            g = (f + o) % FIELD
            pidx[o - 1, f] = pos[min(f, g), max(f, g)]
            mask[o - 1, f] = 0.5 if o == NOFF else 1.0
    return pidx, mask


_PIDX, _PMASK = _pair_index_table()

_EMAT = np.zeros((FLATW, 32), dtype=np.float32)
for _f in range(FIELD):
    _EMAT[_f * WIDE:(_f + 1) * WIDE, _f] = 1.0

_EXPC = np.ascontiguousarray(_EMAT.T)


def kernel(x, beta, arch, embedding, W1, b1, W2, b2, W3, b3, Wo, bo):
    batch = x.shape[0]
    nsplit = 1
    bsp = batch // nsplit
    info = plsc.get_sparse_core_info()
    nw = info.num_cores * info.num_subcores
    table = jnp.concatenate(
        [embedding, jnp.zeros_like(embedding)], axis=1)
    sc = _make_sc_gather(bsp * FIELD)
    xvs = []
    for h in range(nsplit):
        xh = x[h * bsp:(h + 1) * bsp]
        idx3d = xh.reshape(nw, bsp * FIELD // (nw * CHUNK), CHUNK).astype(jnp.int32)
        xvs.append(sc(table, idx3d).reshape(bsp, FLATW))

    ab = jnp.full((1, 32), -1e30, jnp.float32)
    ab = ab.at[0, :FIELD].set((arch / beta).astype(jnp.float32))
    w1a = jnp.zeros((FLATW, W1.shape[1]), jnp.float32)
    w1a = w1a.at[(np.arange(EMBED_OUT) // LAT) * WIDE
                 + (np.arange(EMBED_OUT) % LAT)].set(W1[:EMBED_OUT])
    v = jnp.take(W1, EMBED_OUT + _PIDX.reshape(-1), axis=0)
    v = (v * _PMASK.reshape(-1, 1)).reshape(NOFF, 32, W1.shape[1])

    outs = [
        _mlp_call(
            ab, xv2d, jnp.asarray(_EXPC), jnp.asarray(_EMAT),
            w1a, v, b1.reshape(1, -1),
            W2, b2.reshape(1, -1),
            W3, b3.reshape(1, -1),
            Wo, bo.reshape(1, 1))
        for xv2d in xvs]
    return jnp.concatenate(outs, axis=0)[:, 0]


# speedup vs baseline: 1.4430x; 1.2615x over previous
import functools

import jax
import jax.numpy as jnp
import numpy as np
from jax import lax
from jax.experimental import pallas as pl
from jax.experimental.pallas import tpu as pltpu
from jax.experimental.pallas import tpu_sc as plsc

FIELD = 26
LAT = 64
WIDE = 128
FLATW = FIELD * WIDE
EMBED_OUT = FIELD * LAT
PAIR = FIELD * (FIELD - 1) // 2
NOFF = FIELD // 2
BB = 512
CHUNK = 128


def _make_sc_gather(n_rows):
    info = plsc.get_sparse_core_info()
    nw = info.num_cores * info.num_subcores
    chunks_w = n_rows // (nw * CHUNK)
    half = chunks_w // 2
    mesh = plsc.VectorSubcoreMesh(core_axis_name="c", subcore_axis_name="s")

    @functools.partial(
        pl.kernel, mesh=mesh,
        out_type=jax.ShapeDtypeStruct((n_rows, WIDE), jnp.float32),
        scratch_types=[
            pltpu.VMEM((chunks_w, CHUNK), jnp.int32),
            pltpu.VMEM((2, CHUNK, WIDE), jnp.float32),
            pltpu.SemaphoreType.DMA,
            pltpu.SemaphoreType.DMA,
        ],
    )
    def sc_gather(table_hbm, idx_hbm, out_hbm, idx_v, rows_v, g0, g1):
        wid = lax.axis_index("s") * info.num_cores + lax.axis_index("c")
        rbase = wid * chunks_w * CHUNK
        pltpu.sync_copy(idx_hbm.at[wid], idx_v)

        def start(j, slot, sem):
            pltpu.async_copy(table_hbm.at[idx_v.at[j]], rows_v.at[slot], sem)

        def wait(slot, sem):
            pltpu.make_async_copy(
                table_hbm.at[idx_v.at[0]], rows_v.at[slot], sem).wait()

        def store(j, slot):
            pltpu.sync_copy(rows_v.at[slot],
                            out_hbm.at[pl.ds(rbase + j * CHUNK, CHUNK)])

        start(0, 0, g0)

        def body(g, carry):
            j0 = 2 * g
            start(j0 + 1, 1, g1)
            wait(0, g0)
            store(j0, 0)

            @pl.when(j0 + 2 < chunks_w)
            def _():
                start(j0 + 2, 0, g0)

            wait(1, g1)
            store(j0 + 1, 1)
            return carry

        lax.fori_loop(0, half, body, 0)
        if chunks_w % 2:
            wait(0, g0)
            store(chunks_w - 1, 0)

    return sc_gather


def _mlp_body(ab_ref, xv_ref, expc_ref, e_ref, w1a_ref, v_ref, b1_ref,
              w2_ref, b2_ref, w3_ref, b3_ref, wo_ref, bo_ref, out_ref):
    a = ab_ref[...]
    m = jnp.max(a)
    e = jnp.exp(a - m)
    p = e / jnp.sum(e)
    probrep = jnp.dot(p, expc_ref[...], preferred_element_type=jnp.float32)
    xw = xv_ref[...]
    xc = jnp.concatenate(
        [xw[:, f * WIDE:f * WIDE + LAT] for f in range(FIELD)], axis=1)
    xe = xc * probrep
    acc = jnp.dot(xe, w1a_ref[...], preferred_element_type=jnp.float32)
    for o in range(1, NOFF + 1):
        s = LAT * o
        ro = jnp.concatenate([xe[:, s:], xe[:, :s]], axis=1)
        po = xe * ro
        q = jnp.dot(po, e_ref[...], preferred_element_type=jnp.float32)
        acc = acc + jnp.dot(q, v_ref[o - 1],
                            preferred_element_type=jnp.float32)
    h = jnp.maximum(acc + b1_ref[...], 0.0)
    h = jnp.maximum(
        jnp.dot(h, w2_ref[...], preferred_element_type=jnp.float32)
        + b2_ref[...], 0.0)
    h = jnp.maximum(
        jnp.dot(h, w3_ref[...], preferred_element_type=jnp.float32)
        + b3_ref[...], 0.0)
    out_ref[...] = jnp.dot(h, wo_ref[...],
                           preferred_element_type=jnp.float32) + bo_ref[...]


def _mlp_call(ab, xv2d, expc, ematc, w1a, v, b1, W2, b2, W3, b3, Wo, bo,
              *, interpret=False):
    batch = xv2d.shape[0]
    grid = (batch // BB,)
    full = lambda shape: pl.BlockSpec(shape, lambda i: (0,) * len(shape))
    return pl.pallas_call(
        _mlp_body,
        grid=grid,
        in_specs=[
            full((1, 32)),
            pl.BlockSpec((BB, FLATW), lambda i: (i, 0)),
            full(expc.shape), full(ematc.shape), full(w1a.shape),
            full(v.shape), full((1, b1.shape[1])),
            full(W2.shape), full((1, W2.shape[1])),
            full(W3.shape), full((1, W3.shape[1])),
            full(Wo.shape), full((1, 1)),
        ],
        out_specs=pl.BlockSpec((BB, 1), lambda i: (i, 0)),
        out_shape=jax.ShapeDtypeStruct((batch, 1), jnp.float32),
        interpret=interpret,
    )(ab, xv2d, expc, ematc, w1a, v, b1, W2, b2, W3, b3, Wo, bo)


def _pair_index_table():
    pos = np.zeros((FIELD, FIELD), dtype=np.int32)
    rows, cols = np.triu_indices(FIELD, k=1)
    for k, (i, j) in enumerate(zip(rows, cols)):
        pos[i, j] = k
        pos[j, i] = k
    pidx = np.zeros((NOFF, 32), dtype=np.int32)
    mask = np.zeros((NOFF, 32), dtype=np.float32)
    for o in range(1, NOFF + 1):
        for f in range(FIELD):
            g = (f + o) % FIELD
            pidx[o - 1, f] = pos[min(f, g), max(f, g)]
            mask[o - 1, f] = 0.5 if o == NOFF else 1.0
    return pidx, mask


_PIDX, _PMASK = _pair_index_table()

_EMAT = np.zeros((EMBED_OUT, 32), dtype=np.float32)
for _f in range(FIELD):
    _EMAT[_f * LAT:(_f + 1) * LAT, _f] = 1.0

_EXPC = np.ascontiguousarray(_EMAT.T)


def kernel(x, beta, arch, embedding, W1, b1, W2, b2, W3, b3, Wo, bo):
    batch = x.shape[0]
    nsplit = 1
    bsp = batch // nsplit
    info = plsc.get_sparse_core_info()
    nw = info.num_cores * info.num_subcores
    table = jnp.concatenate(
        [embedding, jnp.zeros_like(embedding)], axis=1)
    sc = _make_sc_gather(bsp * FIELD)
    xvs = []
    for h in range(nsplit):
        xh = x[h * bsp:(h + 1) * bsp]
        idx3d = xh.reshape(nw, bsp * FIELD // (nw * CHUNK), CHUNK).astype(jnp.int32)
        xvs.append(sc(table, idx3d).reshape(bsp, FLATW))

    ab = jnp.full((1, 32), -1e30, jnp.float32)
    ab = ab.at[0, :FIELD].set((arch / beta).astype(jnp.float32))
    w1a = W1[:EMBED_OUT]
    v = jnp.take(W1, EMBED_OUT + _PIDX.reshape(-1), axis=0)
    v = (v * _PMASK.reshape(-1, 1)).reshape(NOFF, 32, W1.shape[1])

    outs = [
        _mlp_call(
            ab, xv2d, jnp.asarray(_EXPC), jnp.asarray(_EMAT),
            w1a, v, b1.reshape(1, -1),
            W2, b2.reshape(1, -1),
            W3, b3.reshape(1, -1),
            Wo, bo.reshape(1, 1))
        for xv2d in xvs]
    return jnp.concatenate(outs, axis=0)[:, 0]


# speedup vs baseline: 1.4713x; 1.0196x over previous
import functools

import jax
import jax.numpy as jnp
import numpy as np
from jax import lax
from jax.experimental import pallas as pl
from jax.experimental.pallas import tpu as pltpu
from jax.experimental.pallas import tpu_sc as plsc

FIELD = 26
LAT = 64
WIDE = 128
FLATW = FIELD * WIDE
EMBED_OUT = FIELD * LAT
PAIR = FIELD * (FIELD - 1) // 2
NOFF = FIELD // 2
BB = 1024
CHUNK = 128


def _make_sc_gather(n_rows):
    info = plsc.get_sparse_core_info()
    nw = info.num_cores * info.num_subcores
    chunks_w = n_rows // (nw * CHUNK)
    half = chunks_w // 2
    mesh = plsc.VectorSubcoreMesh(core_axis_name="c", subcore_axis_name="s")

    @functools.partial(
        pl.kernel, mesh=mesh,
        out_type=jax.ShapeDtypeStruct((n_rows, WIDE), jnp.float32),
        scratch_types=[
            pltpu.VMEM((chunks_w, CHUNK), jnp.int32),
            pltpu.VMEM((2, CHUNK, WIDE), jnp.float32),
            pltpu.SemaphoreType.DMA,
            pltpu.SemaphoreType.DMA,
        ],
    )
    def sc_gather(table_hbm, idx_hbm, out_hbm, idx_v, rows_v, g0, g1):
        wid = lax.axis_index("s") * info.num_cores + lax.axis_index("c")
        rbase = wid * chunks_w * CHUNK
        pltpu.sync_copy(idx_hbm.at[wid], idx_v)

        def start(j, slot, sem):
            pltpu.async_copy(table_hbm.at[idx_v.at[j]], rows_v.at[slot], sem)

        def wait(slot, sem):
            pltpu.make_async_copy(
                table_hbm.at[idx_v.at[0]], rows_v.at[slot], sem).wait()

        def store(j, slot):
            pltpu.sync_copy(rows_v.at[slot],
                            out_hbm.at[pl.ds(rbase + j * CHUNK, CHUNK)])

        start(0, 0, g0)

        def body(g, carry):
            j0 = 2 * g
            start(j0 + 1, 1, g1)
            wait(0, g0)
            store(j0, 0)

            @pl.when(j0 + 2 < chunks_w)
            def _():
                start(j0 + 2, 0, g0)

            wait(1, g1)
            store(j0 + 1, 1)
            return carry

        lax.fori_loop(0, half, body, 0)
        if chunks_w % 2:
            wait(0, g0)
            store(chunks_w - 1, 0)

    return sc_gather


def _mlp_body(ab_ref, xv_ref, expc_ref, e_ref, w1a_ref, v_ref, b1_ref,
              w2_ref, b2_ref, w3_ref, b3_ref, wo_ref, bo_ref, out_ref):
    a = ab_ref[...]
    m = jnp.max(a)
    e = jnp.exp(a - m)
    p = e / jnp.sum(e)
    probrep = jnp.dot(p, expc_ref[...], preferred_element_type=jnp.float32)
    xw = xv_ref[...]
    xc = jnp.concatenate(
        [xw[:, f * WIDE:f * WIDE + LAT] for f in range(FIELD)], axis=1)
    xe = xc * probrep
    acc = jnp.dot(xe.astype(jnp.bfloat16), w1a_ref[...],
                  preferred_element_type=jnp.float32)
    for o in range(1, NOFF + 1):
        s = LAT * o
        ro = jnp.concatenate([xe[:, s:], xe[:, :s]], axis=1)
        po = (xe * ro).astype(jnp.bfloat16)
        q = jnp.dot(po, e_ref[...], preferred_element_type=jnp.float32)
        acc = acc + jnp.dot(q.astype(jnp.bfloat16), v_ref[o - 1],
                            preferred_element_type=jnp.float32)
    h = jnp.maximum(acc + b1_ref[...], 0.0)
    h = jnp.maximum(
        jnp.dot(h, w2_ref[...], preferred_element_type=jnp.float32)
        + b2_ref[...], 0.0)
    h = jnp.maximum(
        jnp.dot(h, w3_ref[...], preferred_element_type=jnp.float32)
        + b3_ref[...], 0.0)
    out_ref[...] = jnp.dot(h, wo_ref[...],
                           preferred_element_type=jnp.float32) + bo_ref[...]


def _mlp_call(ab, xv2d, expc, ematc, w1a, v, b1, W2, b2, W3, b3, Wo, bo,
              *, interpret=False):
    batch = xv2d.shape[0]
    grid = (batch // BB,)
    full = lambda shape: pl.BlockSpec(shape, lambda i: (0,) * len(shape))
    return pl.pallas_call(
        _mlp_body,
        grid=grid,
        in_specs=[
            full((1, 32)),
            pl.BlockSpec((BB, FLATW), lambda i: (i, 0)),
            full(expc.shape), full(ematc.shape), full(w1a.shape),
            full(v.shape), full((1, b1.shape[1])),
            full(W2.shape), full((1, W2.shape[1])),
            full(W3.shape), full((1, W3.shape[1])),
            full(Wo.shape), full((1, 1)),
        ],
        out_specs=pl.BlockSpec((BB, 1), lambda i: (i, 0)),
        out_shape=jax.ShapeDtypeStruct((batch, 1), jnp.float32),
        interpret=interpret,
    )(ab, xv2d, expc, ematc, w1a, v, b1, W2, b2, W3, b3, Wo, bo)


def _pair_index_table():
    pos = np.zeros((FIELD, FIELD), dtype=np.int32)
    rows, cols = np.triu_indices(FIELD, k=1)
    for k, (i, j) in enumerate(zip(rows, cols)):
        pos[i, j] = k
        pos[j, i] = k
    pidx = np.zeros((NOFF, 32), dtype=np.int32)
    mask = np.zeros((NOFF, 32), dtype=np.float32)
    for o in range(1, NOFF + 1):
        for f in range(FIELD):
            g = (f + o) % FIELD
            pidx[o - 1, f] = pos[min(f, g), max(f, g)]
            mask[o - 1, f] = 0.5 if o == NOFF else 1.0
    return pidx, mask


_PIDX, _PMASK = _pair_index_table()

_EMAT = np.zeros((EMBED_OUT, 32), dtype=np.float32)
for _f in range(FIELD):
    _EMAT[_f * LAT:(_f + 1) * LAT, _f] = 1.0

_EXPC = np.ascontiguousarray(_EMAT.T)


def kernel(x, beta, arch, embedding, W1, b1, W2, b2, W3, b3, Wo, bo):
    batch = x.shape[0]
    nsplit = 1
    bsp = batch // nsplit
    info = plsc.get_sparse_core_info()
    nw = info.num_cores * info.num_subcores
    table = jnp.concatenate(
        [embedding, jnp.zeros_like(embedding)], axis=1)
    sc = _make_sc_gather(bsp * FIELD)
    xvs = []
    for h in range(nsplit):
        xh = x[h * bsp:(h + 1) * bsp]
        idx3d = xh.reshape(nw, bsp * FIELD // (nw * CHUNK), CHUNK).astype(jnp.int32)
        xvs.append(sc(table, idx3d).reshape(bsp, FLATW))

    ab = jnp.full((1, 32), -1e30, jnp.float32)
    ab = ab.at[0, :FIELD].set((arch / beta).astype(jnp.float32))
    w1a = W1[:EMBED_OUT]
    v = jnp.take(W1, EMBED_OUT + _PIDX.reshape(-1), axis=0)
    v = (v * _PMASK.reshape(-1, 1)).reshape(NOFF, 32, W1.shape[1])

    outs = [
        _mlp_call(
            ab, xv2d, jnp.asarray(_EXPC), jnp.asarray(_EMAT, jnp.bfloat16),
            w1a.astype(jnp.bfloat16), v.astype(jnp.bfloat16), b1.reshape(1, -1),
            W2, b2.reshape(1, -1),
            W3, b3.reshape(1, -1),
            Wo, bo.reshape(1, 1))
        for xv2d in xvs]
    return jnp.concatenate(outs, axis=0)[:, 0]


# speedup vs baseline: 1.5966x; 1.0852x over previous
import functools

import jax
import jax.numpy as jnp
import numpy as np
from jax import lax
from jax.experimental import pallas as pl
from jax.experimental.pallas import tpu as pltpu
from jax.experimental.pallas import tpu_sc as plsc

FIELD = 26
LAT = 64
WIDE = 128
FLATW = FIELD * WIDE
EMBED_OUT = FIELD * LAT
PAIR = FIELD * (FIELD - 1) // 2
NOFF = FIELD // 2
BB = 1024
CHUNK = 128


def _make_sc_gather(n_rows):
    info = plsc.get_sparse_core_info()
    nw = info.num_cores * info.num_subcores
    chunks_w = n_rows // (nw * CHUNK)
    half = chunks_w // 2
    mesh = plsc.VectorSubcoreMesh(core_axis_name="c", subcore_axis_name="s")

    @functools.partial(
        pl.kernel, mesh=mesh,
        out_type=jax.ShapeDtypeStruct((n_rows, WIDE), jnp.float32),
        scratch_types=[
            pltpu.VMEM((chunks_w, CHUNK), jnp.int32),
            pltpu.VMEM((2, CHUNK, WIDE), jnp.float32),
            pltpu.SemaphoreType.DMA,
            pltpu.SemaphoreType.DMA,
        ],
    )
    def sc_gather(table_hbm, idx_hbm, out_hbm, idx_v, rows_v, g0, g1):
        wid = lax.axis_index("s") * info.num_cores + lax.axis_index("c")
        rbase = wid * chunks_w * CHUNK
        pltpu.sync_copy(idx_hbm.at[wid], idx_v)

        def start(j, slot, sem):
            pltpu.async_copy(table_hbm.at[idx_v.at[j]], rows_v.at[slot], sem)

        def wait(slot, sem):
            pltpu.make_async_copy(
                table_hbm.at[idx_v.at[0]], rows_v.at[slot], sem).wait()

        def store(j, slot):
            pltpu.sync_copy(rows_v.at[slot],
                            out_hbm.at[pl.ds(rbase + j * CHUNK, CHUNK)])

        start(0, 0, g0)

        def body(g, carry):
            j0 = 2 * g
            start(j0 + 1, 1, g1)
            wait(0, g0)
            store(j0, 0)

            @pl.when(j0 + 2 < chunks_w)
            def _():
                start(j0 + 2, 0, g0)

            wait(1, g1)
            store(j0 + 1, 1)
            return carry

        lax.fori_loop(0, half, body, 0)
        if chunks_w % 2:
            wait(0, g0)
            store(chunks_w - 1, 0)

    return sc_gather


def _mlp_body(ab_ref, xv_ref, expc_ref, e_ref, w1a_ref, v_ref, b1_ref,
              w2_ref, b2_ref, w3_ref, b3_ref, wo_ref, bo_ref, out_ref):
    a = ab_ref[...]
    m = jnp.max(a)
    e = jnp.exp(a - m)
    p = e / jnp.sum(e)
    probrep = jnp.dot(p, expc_ref[...], preferred_element_type=jnp.float32)
    xw = xv_ref[...]
    xc = jnp.concatenate(
        [xw[:, f * WIDE:f * WIDE + LAT] for f in range(FIELD)], axis=1)
    xe = xc * probrep
    acc = jnp.dot(xe.astype(jnp.bfloat16), w1a_ref[...],
                  preferred_element_type=jnp.float32)
    qs = []
    for o in range(1, NOFF + 1):
        s = LAT * o
        ro = jnp.concatenate([xe[:, s:], xe[:, :s]], axis=1)
        po = (xe * ro).astype(jnp.bfloat16)
        qs.append(jnp.dot(po, e_ref[...], preferred_element_type=jnp.float32))
    q_all = jnp.concatenate(qs, axis=1).astype(jnp.bfloat16)
    acc = acc + jnp.dot(q_all, v_ref[...], preferred_element_type=jnp.float32)
    h = jnp.maximum(acc + b1_ref[...], 0.0)
    h = jnp.maximum(
        jnp.dot(h, w2_ref[...], preferred_element_type=jnp.float32)
        + b2_ref[...], 0.0)
    h = jnp.maximum(
        jnp.dot(h, w3_ref[...], preferred_element_type=jnp.float32)
        + b3_ref[...], 0.0)
    out_ref[...] = jnp.dot(h, wo_ref[...],
                           preferred_element_type=jnp.float32) + bo_ref[...]


def _mlp_call(ab, xv2d, expc, ematc, w1a, v, b1, W2, b2, W3, b3, Wo, bo,
              *, interpret=False):
    batch = xv2d.shape[0]
    grid = (batch // BB,)
    full = lambda shape: pl.BlockSpec(shape, lambda i: (0,) * len(shape))
    return pl.pallas_call(
        _mlp_body,
        grid=grid,
        in_specs=[
            full((1, 32)),
            pl.BlockSpec((BB, FLATW), lambda i: (i, 0)),
            full(expc.shape), full(ematc.shape), full(w1a.shape),
            full(v.shape), full((1, b1.shape[1])),
            full(W2.shape), full((1, W2.shape[1])),
            full(W3.shape), full((1, W3.shape[1])),
            full(Wo.shape), full((1, 1)),
        ],
        out_specs=pl.BlockSpec((BB, 1), lambda i: (i, 0)),
        out_shape=jax.ShapeDtypeStruct((batch, 1), jnp.float32),
        interpret=interpret,
    )(ab, xv2d, expc, ematc, w1a, v, b1, W2, b2, W3, b3, Wo, bo)


def _pair_index_table():
    pos = np.zeros((FIELD, FIELD), dtype=np.int32)
    rows, cols = np.triu_indices(FIELD, k=1)
    for k, (i, j) in enumerate(zip(rows, cols)):
        pos[i, j] = k
        pos[j, i] = k
    pidx = np.zeros((NOFF, 32), dtype=np.int32)
    mask = np.zeros((NOFF, 32), dtype=np.float32)
    for o in range(1, NOFF + 1):
        for f in range(FIELD):
            g = (f + o) % FIELD
            pidx[o - 1, f] = pos[min(f, g), max(f, g)]
            mask[o - 1, f] = 0.5 if o == NOFF else 1.0
    return pidx, mask


_PIDX, _PMASK = _pair_index_table()

_EMAT = np.zeros((EMBED_OUT, 32), dtype=np.float32)
for _f in range(FIELD):
    _EMAT[_f * LAT:(_f + 1) * LAT, _f] = 1.0

_EXPC = np.ascontiguousarray(_EMAT.T)


def kernel(x, beta, arch, embedding, W1, b1, W2, b2, W3, b3, Wo, bo):
    batch = x.shape[0]
    nsplit = 1
    bsp = batch // nsplit
    info = plsc.get_sparse_core_info()
    nw = info.num_cores * info.num_subcores
    table = jnp.concatenate(
        [embedding, jnp.zeros_like(embedding)], axis=1)
    sc = _make_sc_gather(bsp * FIELD)
    xvs = []
    for h in range(nsplit):
        xh = x[h * bsp:(h + 1) * bsp]
        idx3d = xh.reshape(nw, bsp * FIELD // (nw * CHUNK), CHUNK).astype(jnp.int32)
        xvs.append(sc(table, idx3d).reshape(bsp, FLATW))

    ab = jnp.full((1, 32), -1e30, jnp.float32)
    ab = ab.at[0, :FIELD].set((arch / beta).astype(jnp.float32))
    w1a = W1[:EMBED_OUT]
    v = jnp.take(W1, EMBED_OUT + _PIDX.reshape(-1), axis=0)
    v = (v * _PMASK.reshape(-1, 1)).reshape(NOFF * 32, W1.shape[1])

    outs = [
        _mlp_call(
            ab, xv2d, jnp.asarray(_EXPC), jnp.asarray(_EMAT, jnp.bfloat16),
            w1a.astype(jnp.bfloat16), v.astype(jnp.bfloat16), b1.reshape(1, -1),
            W2, b2.reshape(1, -1),
            W3, b3.reshape(1, -1),
            Wo, bo.reshape(1, 1))
        for xv2d in xvs]
    return jnp.concatenate(outs, axis=0)[:, 0]
